# baseline (speedup 1.0000x reference)
import functools

import jax
import jax.numpy as jnp
import numpy as np
from jax import lax
from jax.experimental import pallas as pl
from jax.experimental.pallas import tpu as pltpu
from jax.experimental.pallas import tpu_sc as plsc


def _proj_body(nf_ref, ws_ref, wd_ref, b1_ref, tsrc_ref, tdst_ref):
    nf = nf_ref[...]
    dn = (((1,), (1,)), ((), ()))
    tsrc_ref[...] = lax.dot_general(nf, ws_ref[...], dn,
                                    preferred_element_type=jnp.float32)
    tdst_ref[...] = lax.dot_general(nf, wd_ref[...], dn,
                                    preferred_element_type=jnp.float32) + b1_ref[...]


def _node_proj(node_feats, w_src, w_dst, b1):
    n, d = node_feats.shape
    hidden = w_src.shape[0]
    blk = 1000
    grid = n // blk
    return pl.pallas_call(
        _proj_body,
        grid=(grid,),
        in_specs=[
            pl.BlockSpec((blk, d), lambda i: (i, 0)),
            pl.BlockSpec((hidden, d), lambda i: (0, 0)),
            pl.BlockSpec((hidden, d), lambda i: (0, 0)),
            pl.BlockSpec((1, hidden), lambda i: (0, 0)),
        ],
        out_specs=[
            pl.BlockSpec((blk, hidden), lambda i: (i, 0)),
            pl.BlockSpec((blk, hidden), lambda i: (i, 0)),
        ],
        out_shape=[
            jax.ShapeDtypeStruct((n, hidden), jnp.float32),
            jax.ShapeDtypeStruct((n, hidden), jnp.float32),
        ],
    )(node_feats, w_src, w_dst, b1.reshape(1, hidden))


def _make_gather_sum(n_edges, hidden, chunk, edge_off, dst_row_off, nbuf=4):
    info = plsc.get_sparse_core_info()
    nw = info.num_cores * info.num_subcores
    per_w = n_edges // nw
    n_chunks = per_w // chunk
    assert per_w % chunk == 0 and chunk % 8 == 0 and per_w % 8 == 0
    assert n_chunks >= nbuf
    n_outer = (n_chunks + nbuf - 1) // nbuf
    ngrp = hidden // 32

    mesh = plsc.VectorSubcoreMesh(core_axis_name="c", subcore_axis_name="s")

    scratch = (
        [pltpu.VMEM((per_w,), jnp.int32)] * 2
        + [pltpu.VMEM((chunk, hidden), jnp.float32)] * (2 * nbuf)
        + [pltpu.VMEM((chunk * hidden,), jnp.bfloat16)] * nbuf
        + [pltpu.SemaphoreType.DMA] * (3 * nbuf)
    )

    @functools.partial(
        pl.kernel,
        mesh=mesh,
        out_type=jax.ShapeDtypeStruct((n_edges * hidden,), jnp.bfloat16),
        scratch_types=scratch,
        compiler_params=pltpu.CompilerParams(needs_layout_passes=False),
    )
    def gather_sum(tsrc_hbm, tdst_hbm, ei_hbm, out_hbm, *sc):
        idxa_v, idxb_v = sc[0], sc[1]
        bufa = sc[2:2 + nbuf]
        bufb = sc[2 + nbuf:2 + 2 * nbuf]
        wbuf = sc[2 + 2 * nbuf:2 + 3 * nbuf]
        ga = sc[2 + 3 * nbuf:2 + 4 * nbuf]
        gb = sc[2 + 4 * nbuf:2 + 5 * nbuf]
        wsem = sc[2 + 5 * nbuf:2 + 6 * nbuf]

        wid = lax.axis_index("s") * info.num_cores + lax.axis_index("c")
        base = wid * per_w

        pltpu.sync_copy(ei_hbm.at[pl.ds(edge_off + base, per_w)], idxa_v)
        pltpu.sync_copy(ei_hbm.at[pl.ds(dst_row_off + edge_off + base, per_w)],
                        idxb_v)

        def start_gathers(ci, s):
            ia = idxa_v.at[pl.ds(ci * chunk, chunk)]
            ib = idxb_v.at[pl.ds(ci * chunk, chunk)]
            pltpu.async_copy(tsrc_hbm.at[ia], bufa[s], ga[s])
            pltpu.async_copy(tdst_hbm.at[ib], bufb[s], gb[s])

        for s in range(nbuf - 1):
            start_gathers(s, s)

        def outer(gi, carry):
            for b in range(nbuf):
                ci = gi * nbuf + b

                @pl.when(ci < n_chunks)
                def _process(b=b, ci=ci):
                    s = b
                    off = base + ci * chunk

                    ia = idxa_v.at[pl.ds(ci * chunk, chunk)]
                    ib = idxb_v.at[pl.ds(ci * chunk, chunk)]
                    pltpu.make_async_copy(tsrc_hbm.at[ia], bufa[s], ga[s]).wait()
                    pltpu.make_async_copy(tdst_hbm.at[ib], bufb[s], gb[s]).wait()

                    @pl.when(ci >= nbuf)
                    def _drain_write():
                        pltpu.make_async_copy(
                            wbuf[s],
                            out_hbm.at[pl.ds(base * hidden, chunk * hidden)],
                            wsem[s]).wait()

                    @plsc.parallel_loop(0, chunk, unroll=1)
                    def _rows(r):
                        for c in range(ngrp):
                            s0 = pl.ds(c * 32, 16)
                            s1 = pl.ds(c * 32 + 16, 16)
                            a0 = bufa[s][r, s0] + bufb[s][r, s0]
                            a1 = bufa[s][r, s1] + bufb[s][r, s1]
                            wbuf[s][pl.ds(r * hidden + c * 32, 32)] = plsc.pack(
                                a0, a1, format=plsc.PackFormat.INTERLEAVED)

                    cp = ci + nbuf - 1
                    s_pre = (b + nbuf - 1) % nbuf

                    @pl.when(cp < n_chunks)
                    def _prefetch():
                        start_gathers(cp, s_pre)

                    pltpu.async_copy(
                        wbuf[s],
                        out_hbm.at[pl.ds(off * hidden, chunk * hidden)],
                        wsem[s])
            return carry

        lax.fori_loop(0, n_outer, outer, 0)

        for s in range(nbuf):
            pltpu.make_async_copy(
                wbuf[s],
                out_hbm.at[pl.ds(base * hidden, chunk * hidden)],
                wsem[s]).wait()

    return gather_sum


def _mlp_compute(ef_ref, gs_ref, we_ref, wo_ref, bo_ref, out_ref):
    dn = (((1,), (1,)), ((), ()))
    me = lax.dot_general(ef_ref[...].astype(jnp.bfloat16),
                         we_ref[...].astype(jnp.bfloat16), dn,
                         preferred_element_type=jnp.float32)
    h = jnp.maximum(me + gs_ref[...].astype(jnp.float32), 0.0)
    out_ref[...] = lax.dot_general(h.astype(jnp.bfloat16),
                                   wo_ref[...].astype(jnp.bfloat16), dn,
                                   preferred_element_type=jnp.float32) + bo_ref[...]


def _mlp_body_first(ef_ref, gs_ref, we_ref, wo_ref, bo_ref, out_ref):
    _mlp_compute(ef_ref, gs_ref, we_ref, wo_ref, bo_ref, out_ref)


def _mlp_body_chain(prev_ref, ef_ref, gs_ref, we_ref, wo_ref, bo_ref, out_ref):
    del prev_ref
    _mlp_compute(ef_ref, gs_ref, we_ref, wo_ref, bo_ref, out_ref)


def _edge_mlp_slice(prev, edge_feats, gsum_k, w_edge, w_out, b_out, edge_off,
                    blk):
    e, d = edge_feats.shape
    hidden = w_edge.shape[0]
    out_dim = w_out.shape[0]
    e_k = gsum_k.shape[0]
    grid = e_k // blk
    assert e_k % blk == 0 and edge_off % blk == 0
    off = edge_off // blk

    common_in = [
        pl.BlockSpec((blk, d), lambda i: (i + off, 0)),
        pl.BlockSpec((blk, hidden), lambda i: (i, 0)),
        pl.BlockSpec((hidden, d), lambda i: (0, 0)),
        pl.BlockSpec((out_dim, hidden), lambda i: (0, 0)),
        pl.BlockSpec((1, out_dim), lambda i: (0, 0)),
    ]
    common_args = (edge_feats, gsum_k, w_edge, w_out,
                   b_out.reshape(1, out_dim))
    out_spec = pl.BlockSpec((blk, out_dim), lambda i: (i + off, 0))
    out_shape = jax.ShapeDtypeStruct((e, out_dim), jnp.float32)
    if prev is None:
        return pl.pallas_call(
            _mlp_body_first,
            grid=(grid,),
            in_specs=common_in,
            out_specs=out_spec,
            out_shape=out_shape,
        )(*common_args)
    return pl.pallas_call(
        _mlp_body_chain,
        grid=(grid,),
        in_specs=[pl.BlockSpec((blk, out_dim), lambda i: (0, 0))] + common_in,
        out_specs=out_spec,
        out_shape=out_shape,
        input_output_aliases={0: 0},
    )(prev, *common_args)


def kernel(edge_feats, node_feats, edge_index, W_edge, W_src, W_dst, b1, W_out, b_out):
    n_edges = edge_feats.shape[0]
    hidden = W_edge.shape[0]
    ei = edge_index.astype(jnp.int32).reshape(-1)
    tsrc, tdst = _node_proj(node_feats, W_src, W_dst, b1)

    perm = np.arange(hidden).reshape(hidden // 32, 2, 16)
    perm = np.transpose(perm, (0, 2, 1)).reshape(hidden)
    w_edge_p = W_edge[perm, :]
    w_out_p = W_out[:, perm]

    slices = [32000, 96000, 96000, 64000, 32000]
    assert sum(slices) == n_edges
    offs = [0]
    for e_k in slices[:-1]:
        offs.append(offs[-1] + e_k)

    gs = [_make_gather_sum(e_k, hidden, chunk=40, edge_off=off,
                           dst_row_off=n_edges, nbuf=6)(
              tsrc, tdst, ei).reshape(e_k, hidden)
          for e_k, off in zip(slices, offs)]

    out = None
    for k in range(len(slices)):
        out = _edge_mlp_slice(out, edge_feats, gs[k], w_edge_p, w_out_p,
                              b_out, offs[k], blk=8000)
    return out


# speedup vs baseline: 1.0152x; 1.0152x over previous
import functools

import jax
import jax.numpy as jnp
import numpy as np
from jax import lax
from jax.experimental import pallas as pl
from jax.experimental.pallas import tpu as pltpu
from jax.experimental.pallas import tpu_sc as plsc


def _proj_body(nf_ref, ws_ref, wd_ref, b1_ref, tsrc_ref, tdst_ref):
    nf = nf_ref[...]
    dn = (((1,), (1,)), ((), ()))
    tsrc_ref[...] = lax.dot_general(nf, ws_ref[...], dn,
                                    preferred_element_type=jnp.float32)
    tdst_ref[...] = lax.dot_general(nf, wd_ref[...], dn,
                                    preferred_element_type=jnp.float32) + b1_ref[...]


def _node_proj(node_feats, w_src, w_dst, b1):
    n, d = node_feats.shape
    hidden = w_src.shape[0]
    blk = 1000
    grid = n // blk
    return pl.pallas_call(
        _proj_body,
        grid=(grid,),
        in_specs=[
            pl.BlockSpec((blk, d), lambda i: (i, 0)),
            pl.BlockSpec((hidden, d), lambda i: (0, 0)),
            pl.BlockSpec((hidden, d), lambda i: (0, 0)),
            pl.BlockSpec((1, hidden), lambda i: (0, 0)),
        ],
        out_specs=[
            pl.BlockSpec((blk, hidden), lambda i: (i, 0)),
            pl.BlockSpec((blk, hidden), lambda i: (i, 0)),
        ],
        out_shape=[
            jax.ShapeDtypeStruct((n, hidden), jnp.float32),
            jax.ShapeDtypeStruct((n, hidden), jnp.float32),
        ],
    )(node_feats, w_src, w_dst, b1.reshape(1, hidden))


def _make_gather_sum(n_edges, hidden, chunk, edge_off, dst_row_off, nbuf=4):
    info = plsc.get_sparse_core_info()
    nw = info.num_cores * info.num_subcores
    per_w = n_edges // nw
    n_chunks = per_w // chunk
    assert per_w % chunk == 0 and chunk % 8 == 0 and per_w % 8 == 0
    assert n_chunks >= nbuf
    n_outer = (n_chunks + nbuf - 1) // nbuf
    ngrp = hidden // 32

    mesh = plsc.VectorSubcoreMesh(core_axis_name="c", subcore_axis_name="s")

    scratch = (
        [pltpu.VMEM((per_w,), jnp.int32)] * 2
        + [pltpu.VMEM((chunk, hidden), jnp.float32)] * (2 * nbuf)
        + [pltpu.VMEM((chunk * hidden,), jnp.bfloat16)] * nbuf
        + [pltpu.SemaphoreType.DMA] * (3 * nbuf)
    )

    @functools.partial(
        pl.kernel,
        mesh=mesh,
        out_type=jax.ShapeDtypeStruct((n_edges * hidden,), jnp.bfloat16),
        scratch_types=scratch,
        compiler_params=pltpu.CompilerParams(needs_layout_passes=False),
    )
    def gather_sum(tsrc_hbm, tdst_hbm, ei_hbm, out_hbm, *sc):
        idxa_v, idxb_v = sc[0], sc[1]
        bufa = sc[2:2 + nbuf]
        bufb = sc[2 + nbuf:2 + 2 * nbuf]
        wbuf = sc[2 + 2 * nbuf:2 + 3 * nbuf]
        ga = sc[2 + 3 * nbuf:2 + 4 * nbuf]
        gb = sc[2 + 4 * nbuf:2 + 5 * nbuf]
        wsem = sc[2 + 5 * nbuf:2 + 6 * nbuf]

        wid = lax.axis_index("s") * info.num_cores + lax.axis_index("c")
        base = wid * per_w

        pltpu.sync_copy(ei_hbm.at[pl.ds(edge_off + base, per_w)], idxa_v)
        pltpu.sync_copy(ei_hbm.at[pl.ds(dst_row_off + edge_off + base, per_w)],
                        idxb_v)

        def start_gathers(ci, s):
            ia = idxa_v.at[pl.ds(ci * chunk, chunk)]
            ib = idxb_v.at[pl.ds(ci * chunk, chunk)]
            pltpu.async_copy(tsrc_hbm.at[ia], bufa[s], ga[s])
            pltpu.async_copy(tdst_hbm.at[ib], bufb[s], gb[s])

        for s in range(nbuf - 1):
            start_gathers(s, s)

        def outer(gi, carry):
            for b in range(nbuf):
                ci = gi * nbuf + b

                @pl.when(ci < n_chunks)
                def _process(b=b, ci=ci):
                    s = b
                    off = base + ci * chunk

                    ia = idxa_v.at[pl.ds(ci * chunk, chunk)]
                    ib = idxb_v.at[pl.ds(ci * chunk, chunk)]
                    pltpu.make_async_copy(tsrc_hbm.at[ia], bufa[s], ga[s]).wait()
                    pltpu.make_async_copy(tdst_hbm.at[ib], bufb[s], gb[s]).wait()

                    @pl.when(ci >= nbuf)
                    def _drain_write():
                        pltpu.make_async_copy(
                            wbuf[s],
                            out_hbm.at[pl.ds(base * hidden, chunk * hidden)],
                            wsem[s]).wait()

                    @plsc.parallel_loop(0, chunk, unroll=1)
                    def _rows(r):
                        for c in range(ngrp):
                            s0 = pl.ds(c * 32, 16)
                            s1 = pl.ds(c * 32 + 16, 16)
                            a0 = bufa[s][r, s0] + bufb[s][r, s0]
                            a1 = bufa[s][r, s1] + bufb[s][r, s1]
                            wbuf[s][pl.ds(r * hidden + c * 32, 32)] = plsc.pack(
                                a0, a1, format=plsc.PackFormat.INTERLEAVED)

                    cp = ci + nbuf - 1
                    s_pre = (b + nbuf - 1) % nbuf

                    @pl.when(cp < n_chunks)
                    def _prefetch():
                        start_gathers(cp, s_pre)

                    pltpu.async_copy(
                        wbuf[s],
                        out_hbm.at[pl.ds(off * hidden, chunk * hidden)],
                        wsem[s])
            return carry

        lax.fori_loop(0, n_outer, outer, 0)

        for s in range(nbuf):
            pltpu.make_async_copy(
                wbuf[s],
                out_hbm.at[pl.ds(base * hidden, chunk * hidden)],
                wsem[s]).wait()

    return gather_sum


def _mlp_compute(ef_ref, gs_ref, we_ref, wo_ref, bo_ref, out_ref):
    dn = (((1,), (1,)), ((), ()))
    me = lax.dot_general(ef_ref[...].astype(jnp.bfloat16),
                         we_ref[...].astype(jnp.bfloat16), dn,
                         preferred_element_type=jnp.float32)
    h = jnp.maximum(me + gs_ref[...].astype(jnp.float32), 0.0)
    out_ref[...] = lax.dot_general(h.astype(jnp.bfloat16),
                                   wo_ref[...].astype(jnp.bfloat16), dn,
                                   preferred_element_type=jnp.float32) + bo_ref[...]


def _mlp_body_first(ef_ref, gs_ref, we_ref, wo_ref, bo_ref, out_ref):
    _mlp_compute(ef_ref, gs_ref, we_ref, wo_ref, bo_ref, out_ref)


def _mlp_body_chain(prev_ref, ef_ref, gs_ref, we_ref, wo_ref, bo_ref, out_ref):
    del prev_ref
    _mlp_compute(ef_ref, gs_ref, we_ref, wo_ref, bo_ref, out_ref)


def _edge_mlp_slice(prev, edge_feats, gsum_k, w_edge, w_out, b_out, edge_off,
                    blk):
    e, d = edge_feats.shape
    hidden = w_edge.shape[0]
    out_dim = w_out.shape[0]
    e_k = gsum_k.shape[0]
    grid = e_k // blk
    assert e_k % blk == 0 and edge_off % blk == 0
    off = edge_off // blk

    common_in = [
        pl.BlockSpec((blk, d), lambda i: (i + off, 0)),
        pl.BlockSpec((blk, hidden), lambda i: (i, 0)),
        pl.BlockSpec((hidden, d), lambda i: (0, 0)),
        pl.BlockSpec((out_dim, hidden), lambda i: (0, 0)),
        pl.BlockSpec((1, out_dim), lambda i: (0, 0)),
    ]
    common_args = (edge_feats, gsum_k, w_edge, w_out,
                   b_out.reshape(1, out_dim))
    out_spec = pl.BlockSpec((blk, out_dim), lambda i: (i + off, 0))
    out_shape = jax.ShapeDtypeStruct((e, out_dim), jnp.float32)
    if prev is None:
        return pl.pallas_call(
            _mlp_body_first,
            grid=(grid,),
            in_specs=common_in,
            out_specs=out_spec,
            out_shape=out_shape,
        )(*common_args)
    return pl.pallas_call(
        _mlp_body_chain,
        grid=(grid,),
        in_specs=[pl.BlockSpec((blk, out_dim), lambda i: (0, 0))] + common_in,
        out_specs=out_spec,
        out_shape=out_shape,
        input_output_aliases={0: 0},
    )(prev, *common_args)


def kernel(edge_feats, node_feats, edge_index, W_edge, W_src, W_dst, b1, W_out, b_out):
    n_edges = edge_feats.shape[0]
    hidden = W_edge.shape[0]
    ei = edge_index.astype(jnp.int32).reshape(-1)
    tsrc, tdst = _node_proj(node_feats, W_src, W_dst, b1)

    perm = np.arange(hidden).reshape(hidden // 32, 2, 16)
    perm = np.transpose(perm, (0, 2, 1)).reshape(hidden)
    w_edge_p = W_edge[perm, :]
    w_out_p = W_out[:, perm]

    slices = [32000, 96000, 96000, 96000]
    assert sum(slices) == n_edges
    offs = [0]
    for e_k in slices[:-1]:
        offs.append(offs[-1] + e_k)

    gs = [_make_gather_sum(e_k, hidden, chunk=40, edge_off=off,
                           dst_row_off=n_edges, nbuf=6)(
              tsrc, tdst, ei).reshape(e_k, hidden)
          for e_k, off in zip(slices, offs)]

    out = None
    for k in range(len(slices)):
        out = _edge_mlp_slice(out, edge_feats, gs[k], w_edge_p, w_out_p,
                              b_out, offs[k], blk=8000)
    return out


# speedup vs baseline: 1.0225x; 1.0072x over previous
import functools

import jax
import jax.numpy as jnp
import numpy as np
from jax import lax
from jax.experimental import pallas as pl
from jax.experimental.pallas import tpu as pltpu
from jax.experimental.pallas import tpu_sc as plsc


def _proj_body(nf_ref, ws_ref, wd_ref, b1_ref, tsrc_ref, tdst_ref):
    nf = nf_ref[...]
    dn = (((1,), (1,)), ((), ()))
    tsrc_ref[...] = lax.dot_general(nf, ws_ref[...], dn,
                                    preferred_element_type=jnp.float32)
    tdst_ref[...] = lax.dot_general(nf, wd_ref[...], dn,
                                    preferred_element_type=jnp.float32) + b1_ref[...]


def _node_proj(node_feats, w_src, w_dst, b1):
    n, d = node_feats.shape
    hidden = w_src.shape[0]
    blk = 1000
    grid = n // blk
    return pl.pallas_call(
        _proj_body,
        grid=(grid,),
        in_specs=[
            pl.BlockSpec((blk, d), lambda i: (i, 0)),
            pl.BlockSpec((hidden, d), lambda i: (0, 0)),
            pl.BlockSpec((hidden, d), lambda i: (0, 0)),
            pl.BlockSpec((1, hidden), lambda i: (0, 0)),
        ],
        out_specs=[
            pl.BlockSpec((blk, hidden), lambda i: (i, 0)),
            pl.BlockSpec((blk, hidden), lambda i: (i, 0)),
        ],
        out_shape=[
            jax.ShapeDtypeStruct((n, hidden), jnp.float32),
            jax.ShapeDtypeStruct((n, hidden), jnp.float32),
        ],
    )(node_feats, w_src, w_dst, b1.reshape(1, hidden))


def _make_gather_sum(n_edges, hidden, chunk, edge_off, dst_row_off, nbuf=4):
    info = plsc.get_sparse_core_info()
    nw = info.num_cores * info.num_subcores
    per_w = n_edges // nw
    n_chunks = per_w // chunk
    assert per_w % chunk == 0 and chunk % 8 == 0 and per_w % 8 == 0
    assert n_chunks >= nbuf
    n_outer = (n_chunks + nbuf - 1) // nbuf
    ngrp = hidden // 32

    mesh = plsc.VectorSubcoreMesh(core_axis_name="c", subcore_axis_name="s")

    scratch = (
        [pltpu.VMEM((per_w,), jnp.int32)] * 2
        + [pltpu.VMEM((chunk, hidden), jnp.float32)] * (2 * nbuf)
        + [pltpu.VMEM((chunk * hidden,), jnp.bfloat16)] * nbuf
        + [pltpu.SemaphoreType.DMA] * (3 * nbuf)
    )

    @functools.partial(
        pl.kernel,
        mesh=mesh,
        out_type=jax.ShapeDtypeStruct((n_edges * hidden,), jnp.bfloat16),
        scratch_types=scratch,
        compiler_params=pltpu.CompilerParams(needs_layout_passes=False),
    )
    def gather_sum(tsrc_hbm, tdst_hbm, ei_hbm, out_hbm, *sc):
        idxa_v, idxb_v = sc[0], sc[1]
        bufa = sc[2:2 + nbuf]
        bufb = sc[2 + nbuf:2 + 2 * nbuf]
        wbuf = sc[2 + 2 * nbuf:2 + 3 * nbuf]
        ga = sc[2 + 3 * nbuf:2 + 4 * nbuf]
        gb = sc[2 + 4 * nbuf:2 + 5 * nbuf]
        wsem = sc[2 + 5 * nbuf:2 + 6 * nbuf]

        wid = lax.axis_index("s") * info.num_cores + lax.axis_index("c")
        base = wid * per_w

        pltpu.sync_copy(ei_hbm.at[pl.ds(edge_off + base, per_w)], idxa_v)
        pltpu.sync_copy(ei_hbm.at[pl.ds(dst_row_off + edge_off + base, per_w)],
                        idxb_v)

        def start_gathers(ci, s):
            ia = idxa_v.at[pl.ds(ci * chunk, chunk)]
            ib = idxb_v.at[pl.ds(ci * chunk, chunk)]
            pltpu.async_copy(tsrc_hbm.at[ia], bufa[s], ga[s])
            pltpu.async_copy(tdst_hbm.at[ib], bufb[s], gb[s])

        for s in range(nbuf - 1):
            start_gathers(s, s)

        def outer(gi, carry):
            for b in range(nbuf):
                ci = gi * nbuf + b

                @pl.when(ci < n_chunks)
                def _process(b=b, ci=ci):
                    s = b
                    off = base + ci * chunk

                    ia = idxa_v.at[pl.ds(ci * chunk, chunk)]
                    ib = idxb_v.at[pl.ds(ci * chunk, chunk)]
                    pltpu.make_async_copy(tsrc_hbm.at[ia], bufa[s], ga[s]).wait()
                    pltpu.make_async_copy(tdst_hbm.at[ib], bufb[s], gb[s]).wait()

                    @pl.when(ci >= nbuf)
                    def _drain_write():
                        pltpu.make_async_copy(
                            wbuf[s],
                            out_hbm.at[pl.ds(base * hidden, chunk * hidden)],
                            wsem[s]).wait()

                    @plsc.parallel_loop(0, chunk, unroll=1)
                    def _rows(r):
                        for c in range(ngrp):
                            s0 = pl.ds(c * 32, 16)
                            s1 = pl.ds(c * 32 + 16, 16)
                            a0 = bufa[s][r, s0] + bufb[s][r, s0]
                            a1 = bufa[s][r, s1] + bufb[s][r, s1]
                            wbuf[s][pl.ds(r * hidden + c * 32, 32)] = plsc.pack(
                                a0, a1, format=plsc.PackFormat.INTERLEAVED)

                    cp = ci + nbuf - 1
                    s_pre = (b + nbuf - 1) % nbuf

                    @pl.when(cp < n_chunks)
                    def _prefetch():
                        start_gathers(cp, s_pre)

                    pltpu.async_copy(
                        wbuf[s],
                        out_hbm.at[pl.ds(off * hidden, chunk * hidden)],
                        wsem[s])
            return carry

        lax.fori_loop(0, n_outer, outer, 0)

        for s in range(nbuf):
            pltpu.make_async_copy(
                wbuf[s],
                out_hbm.at[pl.ds(base * hidden, chunk * hidden)],
                wsem[s]).wait()

    return gather_sum


def _mlp_compute(ef_ref, gs_ref, we_ref, wo_ref, bo_ref, out_ref):
    dn = (((1,), (1,)), ((), ()))
    me = lax.dot_general(ef_ref[...].astype(jnp.bfloat16),
                         we_ref[...].astype(jnp.bfloat16), dn,
                         preferred_element_type=jnp.float32)
    h = jnp.maximum(me + gs_ref[...].astype(jnp.float32), 0.0)
    out_ref[...] = lax.dot_general(h.astype(jnp.bfloat16),
                                   wo_ref[...].astype(jnp.bfloat16), dn,
                                   preferred_element_type=jnp.float32) + bo_ref[...]


def _mlp_body_first(ef_ref, gs_ref, we_ref, wo_ref, bo_ref, out_ref):
    _mlp_compute(ef_ref, gs_ref, we_ref, wo_ref, bo_ref, out_ref)


def _mlp_body_chain(prev_ref, ef_ref, gs_ref, we_ref, wo_ref, bo_ref, out_ref):
    del prev_ref
    _mlp_compute(ef_ref, gs_ref, we_ref, wo_ref, bo_ref, out_ref)


def _edge_mlp_slice(prev, edge_feats, gsum_k, w_edge, w_out, b_out, edge_off,
                    blk):
    e, d = edge_feats.shape
    hidden = w_edge.shape[0]
    out_dim = w_out.shape[0]
    e_k = gsum_k.shape[0]
    grid = e_k // blk
    assert e_k % blk == 0 and edge_off % blk == 0
    off = edge_off // blk

    common_in = [
        pl.BlockSpec((blk, d), lambda i: (i + off, 0)),
        pl.BlockSpec((blk, hidden), lambda i: (i, 0)),
        pl.BlockSpec((hidden, d), lambda i: (0, 0)),
        pl.BlockSpec((out_dim, hidden), lambda i: (0, 0)),
        pl.BlockSpec((1, out_dim), lambda i: (0, 0)),
    ]
    common_args = (edge_feats, gsum_k, w_edge, w_out,
                   b_out.reshape(1, out_dim))
    out_spec = pl.BlockSpec((blk, out_dim), lambda i: (i + off, 0))
    out_shape = jax.ShapeDtypeStruct((e, out_dim), jnp.float32)
    if prev is None:
        return pl.pallas_call(
            _mlp_body_first,
            grid=(grid,),
            in_specs=common_in,
            out_specs=out_spec,
            out_shape=out_shape,
        )(*common_args)
    return pl.pallas_call(
        _mlp_body_chain,
        grid=(grid,),
        in_specs=[pl.BlockSpec((blk, out_dim), lambda i: (0, 0))] + common_in,
        out_specs=out_spec,
        out_shape=out_shape,
        input_output_aliases={0: 0},
    )(prev, *common_args)


def kernel(edge_feats, node_feats, edge_index, W_edge, W_src, W_dst, b1, W_out, b_out):
    n_edges = edge_feats.shape[0]
    hidden = W_edge.shape[0]
    ei = edge_index.astype(jnp.int32).reshape(-1)
    tsrc, tdst = _node_proj(node_feats, W_src, W_dst, b1)

    perm = np.arange(hidden).reshape(hidden // 32, 2, 16)
    perm = np.transpose(perm, (0, 2, 1)).reshape(hidden)
    w_edge_p = W_edge[perm, :]
    w_out_p = W_out[:, perm]

    slices = [32000, 96000, 96000, 96000]
    assert sum(slices) == n_edges
    offs = [0]
    for e_k in slices[:-1]:
        offs.append(offs[-1] + e_k)

    gs = [_make_gather_sum(e_k, hidden, chunk=40, edge_off=off,
                           dst_row_off=n_edges, nbuf=6)(
              tsrc, tdst, ei).reshape(e_k, hidden)
          for e_k, off in zip(slices, offs)]

    out = None
    for k in range(len(slices)):
        out = _edge_mlp_slice(out, edge_feats, gs[k], w_edge_p, w_out_p,
                              b_out, offs[k], blk=16000)
    return out
